# Initial kernel scaffold; baseline (speedup 1.0000x reference)
import jax
import jax.numpy as jnp
from jax.experimental import pallas as pl


def kernel(row, col, row_emb, col_emb, row_bias, col_bias):
    raise NotImplementedError("write your pallas kernel here")


# speedup vs baseline: 10.2781x; 10.2781x over previous
import functools

import jax
import jax.numpy as jnp
from jax import lax
from jax.experimental import pallas as pl
from jax.experimental.pallas import tpu as pltpu
from jax.experimental.pallas import tpu_sc as plsc

B = 16384
VR = 1000
VC = 1000
D = 64

NC = 2
NS = 16
L = 16
NW = NC * NS
BPW = B // NW
NSEG = 4
SEG = BPW // NSEG
GROUPS = BPW // L
GPS = SEG // L


def _body(row_hbm, col_hbm, remb_hbm, cemb_hbm, rbias_hbm, cbias_hbm, out_hbm,
          ridx_v, cidx_v, rrows_v, crows_v, rbias_v, cbias_v, out_v, sem):
    wid = lax.axis_index("s") * NC + lax.axis_index("c")
    base = wid * BPW

    pltpu.sync_copy(row_hbm.at[wid], ridx_v)
    pltpu.sync_copy(col_hbm.at[wid], cidx_v)

    copies = []
    for j in range(NSEG):
        copies.append(pltpu.async_copy(remb_hbm.at[ridx_v.at[j]], rrows_v.at[j], sem))
        copies.append(pltpu.async_copy(cemb_hbm.at[cidx_v.at[j]], crows_v.at[j], sem))
    pltpu.sync_copy(rbias_hbm, rbias_v)
    pltpu.sync_copy(cbias_hbm, cbias_v)
    for c in copies:
        c.wait()

    lane = lax.iota(jnp.int32, L)

    def group(g, carry):
        seg = g // GPS
        off = (g % GPS) * L
        ridx16 = ridx_v[seg, pl.ds(off, L)]
        cidx16 = cidx_v[seg, pl.ds(off, L)]
        acc = plsc.load_gather(rbias_v, [ridx16]) + plsc.load_gather(cbias_v, [cidx16])
        for j in range(L):
            p = off + j
            v = rrows_v[seg, p, pl.ds(0, L)] * crows_v[seg, p, pl.ds(0, L)]
            for k in range(1, D // L):
                v = v + rrows_v[seg, p, pl.ds(k * L, L)] * crows_v[seg, p, pl.ds(k * L, L)]
            s = jnp.sum(v)
            acc = jnp.where(lane == j, acc + s, acc)
        out_v[pl.ds(g * L, L)] = acc
        return carry

    lax.fori_loop(0, GROUPS, group, 0)

    pltpu.sync_copy(out_v, out_hbm.at[pl.ds(base, BPW)])


_sc_call = functools.partial(
    pl.kernel,
    mesh=plsc.VectorSubcoreMesh(core_axis_name="c", subcore_axis_name="s"),
    out_type=jax.ShapeDtypeStruct((B,), jnp.float32),
    compiler_params=pltpu.CompilerParams(
        needs_layout_passes=False, use_tc_tiling_on_sc=False),
    scratch_types=[
        pltpu.VMEM((NSEG, SEG), jnp.int32),
        pltpu.VMEM((NSEG, SEG), jnp.int32),
        pltpu.VMEM((NSEG, SEG, D), jnp.float32),
        pltpu.VMEM((NSEG, SEG, D), jnp.float32),
        pltpu.VMEM((VR,), jnp.float32),
        pltpu.VMEM((VC,), jnp.float32),
        pltpu.VMEM((BPW,), jnp.float32),
        pltpu.SemaphoreType.DMA,
    ],
)(_body)


def kernel(row, col, row_emb, col_emb, row_bias, col_bias):
    row3 = row.astype(jnp.int32).reshape(NW, NSEG, SEG)
    col3 = col.astype(jnp.int32).reshape(NW, NSEG, SEG)
    rbias = row_bias.reshape(VR)
    cbias = col_bias.reshape(VC)
    out = _sc_call(row3, col3, row_emb, col_emb, rbias, cbias)
    return out.reshape(B, 1)
